# baseline (speedup 1.0000x reference)
import functools

import jax
import jax.numpy as jnp
from jax import lax
from jax.experimental import pallas as pl
from jax.experimental.pallas import tpu as pltpu
from jax.experimental.pallas import tpu_sc as plsc

_NUM_SC = 2
_NUM_SUBCORES = 16
_NUM_WORKERS = _NUM_SC * _NUM_SUBCORES

_VOCAB_BLK = 4096
_NBUF = 6


def _gather_sc(ids_flat, embed_table):
    n_ids = ids_flat.shape[0]
    _, hidden = embed_table.shape
    per_worker = n_ids // _NUM_WORKERS

    mesh = plsc.VectorSubcoreMesh(core_axis_name="c", subcore_axis_name="s")

    @functools.partial(
        pl.kernel,
        mesh=mesh,
        out_type=jax.ShapeDtypeStruct((n_ids, hidden), jnp.float32),
        compiler_params=pltpu.CompilerParams(use_tc_tiling_on_sc=False),
        scratch_types=[
            pltpu.VMEM((per_worker,), jnp.int32),
            pltpu.VMEM((per_worker, hidden), jnp.float32),
            pltpu.SemaphoreType.DMA,
        ],
    )
    def gather_kernel(idx_hbm, table_hbm, out_hbm, idx_v, rows_v, sem):
        wid = lax.axis_index("s") * _NUM_SC + lax.axis_index("c")
        base = wid * per_worker
        pltpu.sync_copy(idx_hbm.at[pl.ds(base, per_worker)], idx_v)
        pltpu.async_copy(table_hbm.at[idx_v], rows_v, sem).wait()
        pltpu.sync_copy(rows_v, out_hbm.at[pl.ds(base, per_worker)])

    return gather_kernel(ids_flat, embed_table)


def _proj_body(nblk, vocab, x_ref, w_ref, b_ref, out_ref, acc_ref, tail_ref,
               sems):
    i = pl.program_id(0)
    slot = lax.rem(i, _NBUF)
    tail = vocab - (nblk - 1) * _VOCAB_BLK

    def _slot_copy(step):
        return pltpu.make_async_copy(
            acc_ref.at[step % _NBUF],
            out_ref.at[:, pl.ds(step * _VOCAB_BLK, _VOCAB_BLK)],
            sems.at[step % _NBUF],
        )

    def _tail_copy():
        return pltpu.make_async_copy(
            tail_ref,
            out_ref.at[:, pl.ds((nblk - 1) * _VOCAB_BLK, tail)],
            sems.at[(nblk - 1) % _NBUF],
        )

    @pl.when(i >= _NBUF)
    def _():
        _slot_copy(i - _NBUF).wait()

    acc = lax.dot_general(
        x_ref[...], w_ref[...],
        (((1,), (1,)), ((), ())),
        preferred_element_type=jnp.float32,
    ) + b_ref[...]

    @pl.when(i < nblk - 1)
    def _():
        acc_ref[slot] = acc
        _slot_copy(i).start()

    @pl.when(i == nblk - 1)
    def _():
        tail_ref[...] = acc[:, :tail]
        _tail_copy().start()
        for back in reversed(range(1, min(_NBUF, nblk))):
            _slot_copy(nblk - 1 - back).wait()
        _tail_copy().wait()


def _project_tc(x, proj_W, proj_b, interpret=False):
    n_tok, hidden = x.shape
    vocab = proj_W.shape[0]
    nblk = pl.cdiv(vocab, _VOCAB_BLK)
    bias2d = jnp.pad(proj_b, (0, nblk * _VOCAB_BLK - vocab)).reshape(1, -1)
    return pl.pallas_call(
        functools.partial(_proj_body, nblk, vocab),
        grid=(nblk,),
        in_specs=[
            pl.BlockSpec((n_tok, hidden), lambda i: (0, 0)),
            pl.BlockSpec((_VOCAB_BLK, hidden), lambda i: (i, 0)),
            pl.BlockSpec((1, _VOCAB_BLK), lambda i: (0, i)),
        ],
        out_specs=pl.BlockSpec(memory_space=pl.ANY),
        out_shape=jax.ShapeDtypeStruct((n_tok, vocab), jnp.float32),
        scratch_shapes=[
            pltpu.VMEM((_NBUF, n_tok, _VOCAB_BLK), jnp.float32),
            pltpu.VMEM((n_tok, vocab - (nblk - 1) * _VOCAB_BLK), jnp.float32),
            pltpu.SemaphoreType.DMA((_NBUF,)),
        ],
        interpret=interpret,
    )(x, proj_W, bias2d)


def kernel(input_ids, embed_table, proj_W, proj_b):
    batch, qlen = input_ids.shape
    vocab, _ = embed_table.shape
    ids_flat = input_ids.reshape(batch * qlen)
    x = _gather_sc(ids_flat, embed_table)
    logits = _project_tc(x, proj_W, proj_b)
    return logits.reshape(batch, qlen, vocab)


# speedup vs baseline: 1.8465x; 1.8465x over previous
import functools

import jax
import jax.numpy as jnp
from jax import lax
from jax.experimental import pallas as pl
from jax.experimental.pallas import tpu as pltpu
from jax.experimental.pallas import tpu_sc as plsc

_NUM_SC = 2
_NUM_SUBCORES = 16
_NUM_WORKERS = _NUM_SC * _NUM_SUBCORES

_VOCAB_BLK = 4096
_QLEN = 4


def _gather_sc(ids_flat, embed_table):
    n_ids = ids_flat.shape[0]
    _, hidden = embed_table.shape
    per_worker = n_ids // _NUM_WORKERS

    mesh = plsc.VectorSubcoreMesh(core_axis_name="c", subcore_axis_name="s")

    @functools.partial(
        pl.kernel,
        mesh=mesh,
        out_type=jax.ShapeDtypeStruct((n_ids, hidden), jnp.float32),
        compiler_params=pltpu.CompilerParams(use_tc_tiling_on_sc=False),
        scratch_types=[
            pltpu.VMEM((per_worker,), jnp.int32),
            pltpu.VMEM((per_worker, hidden), jnp.float32),
            pltpu.SemaphoreType.DMA,
        ],
    )
    def gather_kernel(idx_hbm, table_hbm, out_hbm, idx_v, rows_v, sem):
        wid = lax.axis_index("s") * _NUM_SC + lax.axis_index("c")
        base = wid * per_worker
        pltpu.sync_copy(idx_hbm.at[pl.ds(base, per_worker)], idx_v)
        pltpu.async_copy(table_hbm.at[idx_v], rows_v, sem).wait()
        pltpu.sync_copy(rows_v, out_hbm.at[pl.ds(base, per_worker)])

    return gather_kernel(ids_flat, embed_table)


def _proj_body(x_ref, w_ref, b_ref, out_ref):
    acc = lax.dot_general(
        x_ref[...], w_ref[...],
        (((1,), (1,)), ((), ())),
        preferred_element_type=jnp.float32,
    ) + b_ref[...][None, :]
    out_ref[...] = acc.reshape(out_ref.shape)


def _project_tc(x, proj_W, proj_b, interpret=False):
    n_tok, hidden = x.shape
    vocab = proj_W.shape[0]
    batch = n_tok // _QLEN
    nblk = pl.cdiv(vocab, _VOCAB_BLK)
    return pl.pallas_call(
        _proj_body,
        grid=(nblk,),
        in_specs=[
            pl.BlockSpec((n_tok, hidden), lambda i: (0, 0)),
            pl.BlockSpec((_VOCAB_BLK, hidden), lambda i: (i, 0)),
            pl.BlockSpec((_VOCAB_BLK,), lambda i: (i,)),
        ],
        out_specs=pl.BlockSpec((batch, _QLEN, _VOCAB_BLK), lambda i: (0, 0, i)),
        out_shape=jax.ShapeDtypeStruct((batch, _QLEN, vocab), jnp.float32),
        interpret=interpret,
    )(x, proj_W, proj_b)


def kernel(input_ids, embed_table, proj_W, proj_b):
    batch, qlen = input_ids.shape
    ids_flat = input_ids.reshape(batch * qlen)
    x = _gather_sc(ids_flat, embed_table)
    return _project_tc(x, proj_W, proj_b)


# speedup vs baseline: 1.8567x; 1.0055x over previous
import functools

import jax
import jax.numpy as jnp
from jax import lax
from jax.experimental import pallas as pl
from jax.experimental.pallas import tpu as pltpu
from jax.experimental.pallas import tpu_sc as plsc

_NUM_SC = 2
_NUM_SUBCORES = 16
_NUM_WORKERS = _NUM_SC * _NUM_SUBCORES

_VOCAB_BLK = 4096
_QLEN = 4


def _gather_sc(ids_flat, embed_table):
    n_ids = ids_flat.shape[0]
    _, hidden = embed_table.shape
    per_worker = n_ids // _NUM_WORKERS

    mesh = plsc.VectorSubcoreMesh(core_axis_name="c", subcore_axis_name="s")

    @functools.partial(
        pl.kernel,
        mesh=mesh,
        out_type=jax.ShapeDtypeStruct((n_ids, 128), jnp.float32),
        compiler_params=pltpu.CompilerParams(use_tc_tiling_on_sc=False),
        scratch_types=[
            pltpu.VMEM((per_worker,), jnp.int32),
            pltpu.VMEM((per_worker, hidden), jnp.float32),
            pltpu.SemaphoreType.DMA,
        ],
    )
    def gather_kernel(idx_hbm, table_hbm, out_hbm, idx_v, rows_v, sem):
        wid = lax.axis_index("s") * _NUM_SC + lax.axis_index("c")
        base = wid * per_worker
        pltpu.sync_copy(idx_hbm.at[pl.ds(base, per_worker)], idx_v)
        pltpu.async_copy(table_hbm.at[idx_v], rows_v, sem).wait()
        pltpu.sync_copy(
            rows_v, out_hbm.at[pl.ds(base, per_worker), pl.ds(0, hidden)]
        )

    return gather_kernel(ids_flat, embed_table)


def _proj_body(hidden, x_ref, w_ref, b_ref, out_ref):
    acc = lax.dot_general(
        x_ref[...][:, :hidden], w_ref[...],
        (((1,), (1,)), ((), ())),
        preferred_element_type=jnp.float32,
    ) + b_ref[...][None, :]
    out_ref[...] = acc.reshape(out_ref.shape)


def _project_tc(x, proj_W, proj_b, interpret=False):
    n_tok, x_width = x.shape
    vocab, hidden = proj_W.shape
    batch = n_tok // _QLEN
    nblk = pl.cdiv(vocab, _VOCAB_BLK)
    return pl.pallas_call(
        functools.partial(_proj_body, hidden),
        grid=(nblk,),
        in_specs=[
            pl.BlockSpec((n_tok, x_width), lambda i: (0, 0)),
            pl.BlockSpec((_VOCAB_BLK, hidden), lambda i: (i, 0)),
            pl.BlockSpec((_VOCAB_BLK,), lambda i: (i,)),
        ],
        out_specs=pl.BlockSpec((batch, _QLEN, _VOCAB_BLK), lambda i: (0, 0, i)),
        out_shape=jax.ShapeDtypeStruct((batch, _QLEN, vocab), jnp.float32),
        interpret=interpret,
    )(x, proj_W, proj_b)


def kernel(input_ids, embed_table, proj_W, proj_b):
    batch, qlen = input_ids.shape
    ids_flat = input_ids.reshape(batch * qlen)
    x = _gather_sc(ids_flat, embed_table)
    return _project_tc(x, proj_W, proj_b)


# speedup vs baseline: 2.4455x; 1.3172x over previous
import functools

import jax
import jax.numpy as jnp
from jax import lax
from jax.experimental import pallas as pl
from jax.experimental.pallas import tpu as pltpu
from jax.experimental.pallas import tpu_sc as plsc

_NUM_SC = 2
_NUM_SUBCORES = 16
_NUM_WORKERS = _NUM_SC * _NUM_SUBCORES

_VOCAB_BLK = 4096
_QLEN = 4


def _gather_sc(ids_flat, embed_table):
    n_ids = ids_flat.shape[0]
    _, hidden = embed_table.shape
    per_worker = n_ids // _NUM_WORKERS

    mesh = plsc.VectorSubcoreMesh(core_axis_name="c", subcore_axis_name="s")

    @functools.partial(
        pl.kernel,
        mesh=mesh,
        out_type=jax.ShapeDtypeStruct((n_ids, 128), jnp.float32),
        compiler_params=pltpu.CompilerParams(use_tc_tiling_on_sc=False),
        scratch_types=[
            pltpu.VMEM((per_worker,), jnp.int32),
            pltpu.VMEM((per_worker, hidden), jnp.float32),
            pltpu.SemaphoreType.DMA,
        ],
    )
    def gather_kernel(idx_hbm, table_hbm, out_hbm, idx_v, rows_v, sem):
        wid = lax.axis_index("s") * _NUM_SC + lax.axis_index("c")
        base = wid * per_worker
        pltpu.sync_copy(idx_hbm.at[pl.ds(base, per_worker)], idx_v)
        pltpu.async_copy(table_hbm.at[idx_v], rows_v, sem).wait()
        pltpu.sync_copy(
            rows_v, out_hbm.at[pl.ds(base, per_worker), pl.ds(0, hidden)]
        )

    return gather_kernel(ids_flat, embed_table)


def _proj_body(hidden, x_ref, wt_ref, b_ref, out_ref):
    acc = lax.dot_general(
        x_ref[...][:, :hidden], wt_ref[...],
        (((1,), (0,)), ((), ())),
        preferred_element_type=jnp.float32,
    ) + b_ref[...][None, :]
    out_ref[...] = acc.reshape(out_ref.shape)


def _project_tc(x, proj_W, proj_b, interpret=False):
    n_tok, x_width = x.shape
    vocab, hidden = proj_W.shape
    batch = n_tok // _QLEN
    nblk = pl.cdiv(vocab, _VOCAB_BLK)
    wT = proj_W.T
    return pl.pallas_call(
        functools.partial(_proj_body, hidden),
        grid=(nblk,),
        in_specs=[
            pl.BlockSpec((n_tok, x_width), lambda i: (0, 0)),
            pl.BlockSpec((hidden, _VOCAB_BLK), lambda i: (0, i)),
            pl.BlockSpec((_VOCAB_BLK,), lambda i: (i,)),
        ],
        out_specs=pl.BlockSpec((batch, _QLEN, _VOCAB_BLK), lambda i: (0, 0, i)),
        out_shape=jax.ShapeDtypeStruct((batch, _QLEN, vocab), jnp.float32),
        interpret=interpret,
    )(x, wT, proj_b)


def kernel(input_ids, embed_table, proj_W, proj_b):
    batch, qlen = input_ids.shape
    ids_flat = input_ids.reshape(batch * qlen)
    x = _gather_sc(ids_flat, embed_table)
    return _project_tc(x, proj_W, proj_b)


# speedup vs baseline: 2.5679x; 1.0500x over previous
import functools

import jax
import jax.numpy as jnp
from jax import lax
from jax.experimental import pallas as pl
from jax.experimental.pallas import tpu as pltpu
from jax.experimental.pallas import tpu_sc as plsc

_NUM_SC = 2
_NUM_SUBCORES = 16
_NUM_WORKERS = _NUM_SC * _NUM_SUBCORES

_VOCAB_BLK = 16384
_QLEN = 4


def _gather_sc(ids_flat, embed_table):
    n_ids = ids_flat.shape[0]
    _, hidden = embed_table.shape
    per_worker = n_ids // _NUM_WORKERS

    mesh = plsc.VectorSubcoreMesh(core_axis_name="c", subcore_axis_name="s")

    @functools.partial(
        pl.kernel,
        mesh=mesh,
        out_type=jax.ShapeDtypeStruct((n_ids, 128), jnp.float32),
        compiler_params=pltpu.CompilerParams(use_tc_tiling_on_sc=False),
        scratch_types=[
            pltpu.VMEM((per_worker,), jnp.int32),
            pltpu.VMEM((per_worker, hidden), jnp.float32),
            pltpu.SemaphoreType.DMA,
        ],
    )
    def gather_kernel(idx_hbm, table_hbm, out_hbm, idx_v, rows_v, sem):
        wid = lax.axis_index("s") * _NUM_SC + lax.axis_index("c")
        base = wid * per_worker
        pltpu.sync_copy(idx_hbm.at[pl.ds(base, per_worker)], idx_v)
        pltpu.async_copy(table_hbm.at[idx_v], rows_v, sem).wait()
        pltpu.sync_copy(
            rows_v, out_hbm.at[pl.ds(base, per_worker), pl.ds(0, hidden)]
        )

    return gather_kernel(ids_flat, embed_table)


def _proj_body(hidden, x_ref, wt_ref, b_ref, out_ref):
    acc = lax.dot_general(
        x_ref[...][:, :hidden], wt_ref[...],
        (((1,), (0,)), ((), ())),
        preferred_element_type=jnp.float32,
    ) + b_ref[...][None, :]
    out_ref[...] = acc.reshape(out_ref.shape)


def _project_tc(x, proj_W, proj_b, interpret=False):
    n_tok, x_width = x.shape
    vocab, hidden = proj_W.shape
    batch = n_tok // _QLEN
    nblk = pl.cdiv(vocab, _VOCAB_BLK)
    wT = proj_W.T
    return pl.pallas_call(
        functools.partial(_proj_body, hidden),
        grid=(nblk,),
        in_specs=[
            pl.BlockSpec((n_tok, x_width), lambda i: (0, 0)),
            pl.BlockSpec((hidden, _VOCAB_BLK), lambda i: (0, i)),
            pl.BlockSpec((_VOCAB_BLK,), lambda i: (i,)),
        ],
        out_specs=pl.BlockSpec((batch, _QLEN, _VOCAB_BLK), lambda i: (0, 0, i)),
        out_shape=jax.ShapeDtypeStruct((batch, _QLEN, vocab), jnp.float32),
        interpret=interpret,
    )(x, wT, proj_b)


def kernel(input_ids, embed_table, proj_W, proj_b):
    batch, qlen = input_ids.shape
    ids_flat = input_ids.reshape(batch * qlen)
    x = _gather_sc(ids_flat, embed_table)
    return _project_tc(x, proj_W, proj_b)
